# Initial kernel scaffold; baseline (speedup 1.0000x reference)
import jax
import jax.numpy as jnp
from jax.experimental import pallas as pl


def kernel(x, context, coord4_grid, coord4_ico, params, pool_nbrs, sa_nbrs, dec_nbrs, enc_q_idx, dec_q_idx):
    raise NotImplementedError("write your pallas kernel here")


# speedup vs baseline: 1.0056x; 1.0056x over previous
import functools

import jax
import jax.numpy as jnp
import numpy as np
from jax.experimental import pallas as pl

N0 = 50000
N1 = 10242
CIN = 128
CC = 16
HID = 128
LAT = 32
HEADS = 8
DH = HID // HEADS
SAT_BOUND = 5.0


def _layernorm(x):
    m = x.mean(-1, keepdims=True)
    v = x.var(-1, keepdims=True)
    return (x - m) * jax.lax.rsqrt(v + 1e-5)


def _stage_a_body(x_ref, feat_ref, wce_ref, bce_ref, wk_ref, wv_ref,
                  x0_ref, k_ref, v_ref):
    feat = feat_ref[...]
    x0 = x_ref[...] + jnp.sin(
        jnp.dot(feat, wce_ref[...], preferred_element_type=jnp.float32)
        + bce_ref[...])
    x0_ref[...] = x0
    ln = _layernorm(x0)
    k_ref[...] = jnp.dot(ln, wk_ref[...], preferred_element_type=jnp.float32)
    v_ref[...] = jnp.dot(ln, wv_ref[...], preferred_element_type=jnp.float32)


def _stage_a(x, feat, wce, bce, wk, wv, block):
    n = x.shape[0]
    grid = n // block
    f = feat.shape[1]
    return pl.pallas_call(
        _stage_a_body,
        grid=(grid,),
        in_specs=[
            pl.BlockSpec((block, CIN), lambda i: (i, 0)),
            pl.BlockSpec((block, f), lambda i: (i, 0)),
            pl.BlockSpec((f, CIN), lambda i: (0, 0)),
            pl.BlockSpec((1, CIN), lambda i: (0, 0)),
            pl.BlockSpec((CIN, HID), lambda i: (0, 0)),
            pl.BlockSpec((CIN, HID), lambda i: (0, 0)),
        ],
        out_specs=[
            pl.BlockSpec((block, CIN), lambda i: (i, 0)),
            pl.BlockSpec((block, HID), lambda i: (i, 0)),
            pl.BlockSpec((block, HID), lambda i: (i, 0)),
        ],
        out_shape=[
            jax.ShapeDtypeStruct((n, CIN), jnp.float32),
            jax.ShapeDtypeStruct((n, HID), jnp.float32),
            jax.ShapeDtypeStruct((n, HID), jnp.float32),
        ],
    )(x, feat, wce, bce, wk, wv)


def _nbr_attention_pre(q, kk, vv):
    nq, k, _ = kk.shape
    qh = q.reshape(nq, HEADS, DH)
    kh = kk.reshape(nq, k, HEADS, DH)
    vh = vv.reshape(nq, k, HEADS, DH)
    logits = jnp.einsum("nhd,nkhd->nhk", qh, kh) / float(np.sqrt(DH))
    w = jax.nn.softmax(logits, axis=-1)
    return jnp.einsum("nhk,nkhd->nhd", w, vh).reshape(nq, HID)


def kernel(x, context, coord4_grid, coord4_ico, params, pool_nbrs, sa_nbrs,
           dec_nbrs, enc_q_idx, dec_q_idx):
    feat_grid = jnp.concatenate([coord4_grid, context], axis=-1)
    pe = params["enc_pool"]
    x0, K0, V0 = _stage_a(x, feat_grid, params["Wce0"],
                          params["bce0"].reshape(1, CIN), pe["Wk"], pe["Wv"],
                          block=2000)

    xq0 = jnp.take(x0, enc_q_idx, axis=0)
    q0 = _layernorm(xq0) @ pe["Wq"]
    kk = jnp.take(K0, pool_nbrs, axis=0)
    vv = jnp.take(V0, pool_nbrs, axis=0)
    h = xq0 @ pe["Wskip"] + _nbr_attention_pre(q0, kk, vv) @ pe["Wo"]

    ctx_ico = jnp.take(context, enc_q_idx, axis=0)
    feat_ico = jnp.concatenate([coord4_ico, ctx_ico], axis=-1)
    h = h + jnp.sin(feat_ico @ params["Wce1"] + params["bce1"])

    for p in params["enc_sa"]:
        hn = _layernorm(h)
        kk = jnp.take(hn @ p["Wk"], sa_nbrs, axis=0)
        vv = jnp.take(hn @ p["Wv"], sa_nbrs, axis=0)
        h = h + _nbr_attention_pre(hn @ p["Wq"], kk, vv) @ p["Wo"]

    z = h @ params["Wlat"] + params["blat"]
    z = z * jax.lax.rsqrt(1.0 + (z / SAT_BOUND) ** 2)
    g = z @ params["Wdlat"] + params["bdlat"]
    g = g + jnp.sin(feat_ico @ params["Wcd1"] + params["bcd1"])

    for p in params["dec_sa"]:
        gn = _layernorm(g)
        kk = jnp.take(gn @ p["Wk"], sa_nbrs, axis=0)
        vv = jnp.take(gn @ p["Wv"], sa_nbrs, axis=0)
        g = g + _nbr_attention_pre(gn @ p["Wq"], kk, vv) @ p["Wo"]

    pd = params["dec_pool"]
    gn = _layernorm(g)
    Kd = gn @ pd["Wk"]
    Vd = gn @ pd["Wv"]
    xqd = jnp.take(g, dec_q_idx, axis=0)
    qd = _layernorm(xqd) @ pd["Wq"]
    kk = jnp.take(Kd, dec_nbrs, axis=0)
    vv = jnp.take(Vd, dec_nbrs, axis=0)
    out = xqd @ pd["Wskip"] + _nbr_attention_pre(qd, kk, vv) @ pd["Wo"]
    return out
